# baseline (speedup 1.0000x reference)
import jax
import jax.numpy as jnp
from jax.experimental import pallas as pl
from jax.experimental.pallas import tpu as pltpu

Q = 3136
K = 16384
D = 1536
NN = 9

BQ = 784
RC = 112
BK = 1024
NQB = Q // BQ
NKB = K // BK
NG = BK // 128

BIGF = 3.0e38
MASKHI = -2048

_NET8 = [(0, 1), (2, 3), (4, 5), (6, 7), (0, 2), (1, 3), (4, 6), (5, 7),
         (1, 2), (5, 6), (0, 4), (1, 5), (2, 6), (3, 7), (2, 4), (3, 5),
         (1, 2), (3, 4), (5, 6)]
_NET13 = [(0, 8), (2, 6), (3, 7), (4, 8), (1, 3), (2, 4), (3, 5), (5, 7),
          (6, 8), (1, 2), (3, 4), (5, 6), (7, 8)]


def _knn_body(emb_ref, mem_ref, out_ref, cand_ref, run_ref, norms_ref,
              embbf_ref):
    qi = pl.program_id(0)
    k = pl.program_id(1)

    @pl.when(k == 0)
    def _init():
        run_ref[...] = jnp.full((BQ, NN * 128), BIGF, dtype=jnp.float32)
        embbf_ref[...] = (-2.0 * emb_ref[...]).astype(jnp.bfloat16)

    @pl.when(jnp.logical_and(qi == 0, k < NKB))
    def _norms():
        mem = mem_ref[...]
        norms_ref[k, 0:1, :] = jnp.sum(mem * mem, axis=1)[None, :]

    d = jax.lax.dot_general(
        embbf_ref[...], mem_ref[...].astype(jnp.bfloat16),
        (((1,), (1,)), ((), ())),
        preferred_element_type=jnp.float32)

    def merge_chunk(c, carry):
        rows = pl.ds(c * RC, RC)
        g = [cand_ref[rows, i * 128:(i + 1) * 128] for i in range(NG)]
        for i, j in _NET8:
            lo = jnp.minimum(g[i], g[j])
            hi = jnp.maximum(g[i], g[j])
            g[i], g[j] = lo, hi
        r = [run_ref[rows, i * 128:(i + 1) * 128] for i in range(NN)]
        cc = [r[0]] + [jnp.minimum(r[i], g[NN - 1 - i]) for i in range(1, NN)]
        for i, j in _NET13:
            lo = jnp.minimum(cc[i], cc[j])
            hi = jnp.maximum(cc[i], cc[j])
            cc[i], cc[j] = lo, hi

        @pl.when(k < NKB)
        def _store():
            for i in range(NN):
                run_ref[rows, i * 128:(i + 1) * 128] = cc[i]

        @pl.when(k == NKB)
        def _extract():
            lvl = list(cc)
            outs = []
            for _ in range(NN):
                mn = jnp.min(lvl[0], axis=1, keepdims=True)
                won = lvl[0] == mn
                for i in range(NN - 1):
                    lvl[i] = jnp.where(won, lvl[i + 1], lvl[i])
                lvl[NN - 1] = jnp.where(won, BIGF, lvl[NN - 1])
                outs.append(mn)
            keys = jnp.concatenate(outs, axis=1)
            vb = jax.lax.bitcast_convert_type(keys, jnp.int32) & MASKHI
            vals = jax.lax.bitcast_convert_type(vb, jnp.float32)
            emb = emb_ref[rows, :]
            q_sq = jnp.sum(emb * emb, axis=1, keepdims=True)
            out_ref[rows, :] = jnp.sqrt(jnp.maximum(vals + q_sq, 1e-12))

        return carry

    @pl.when(k > 0)
    def _merge():
        jax.lax.fori_loop(0, BQ // RC, merge_chunk, 0, unroll=True)

    @pl.when(k < NKB)
    def _stash():
        sq = norms_ref[k, 0:1, :] + d
        bits = jax.lax.bitcast_convert_type(sq, jnp.int32)
        col = jax.lax.broadcasted_iota(jnp.int32, (BQ, BK), 1)
        cand_ref[...] = jax.lax.bitcast_convert_type(
            (bits & MASKHI) | col, jnp.float32)


@jax.jit
def kernel(embedding, memory_bank):
    return pl.pallas_call(
        _knn_body,
        grid=(NQB, NKB + 1),
        in_specs=[
            pl.BlockSpec((BQ, D), lambda q, k: (q, 0)),
            pl.BlockSpec((BK, D), lambda q, k: (jnp.minimum(k, NKB - 1), 0)),
        ],
        out_specs=pl.BlockSpec((BQ, NN), lambda q, k: (q, 0)),
        out_shape=jax.ShapeDtypeStruct((Q, NN), jnp.float32),
        scratch_shapes=[
            pltpu.VMEM((BQ, BK), jnp.float32),
            pltpu.VMEM((BQ, NN * 128), jnp.float32),
            pltpu.VMEM((NKB, 8, BK), jnp.float32),
            pltpu.VMEM((BQ, D), jnp.bfloat16),
        ],
        compiler_params=pltpu.CompilerParams(
            dimension_semantics=("arbitrary", "arbitrary")),
    )(embedding, memory_bank)


# speedup vs baseline: 1.0978x; 1.0978x over previous
import jax
import jax.numpy as jnp
from jax.experimental import pallas as pl
from jax.experimental.pallas import tpu as pltpu

Q = 3136
K = 16384
D = 1536
NN = 9

BQ = 784
RC = 112
BK = 1024
NQB = Q // BQ
NKB = K // BK
NG = BK // 128

BIGF = 3.0e38
MASKHI = -2048

_NET8 = [(0, 1), (2, 3), (4, 5), (6, 7), (0, 2), (1, 3), (4, 6), (5, 7),
         (1, 2), (5, 6), (0, 4), (1, 5), (2, 6), (3, 7), (2, 4), (3, 5),
         (1, 2), (3, 4), (5, 6)]
_NET13 = [(0, 8), (2, 6), (3, 7), (4, 8), (1, 3), (2, 4), (3, 5), (5, 7),
          (6, 8), (1, 2), (3, 4), (5, 6), (7, 8)]


def _knn_body(emb_ref, mem_ref, out_ref, cand_ref, run_ref, norms_ref,
              membf_ref, embbf_ref):
    k = pl.program_id(0)
    qi = pl.program_id(1)

    @pl.when(jnp.logical_and(k == 0, qi == 0))
    def _init():
        run_ref[...] = jnp.full((NQB, BQ, NN * 128), BIGF,
                                dtype=jnp.float32)

    @pl.when(qi == 0)
    def _norms():
        mem = mem_ref[...]
        membf_ref[...] = mem.astype(jnp.bfloat16)
        norms_ref[0:1, :] = jnp.sum(mem * mem, axis=1)[None, :]

    @pl.when(k == 0)
    def _embcast():
        embbf_ref[qi] = (-2.0 * emb_ref[...]).astype(jnp.bfloat16)

    d = jax.lax.dot_general(
        embbf_ref[qi], membf_ref[...],
        (((1,), (1,)), ((), ())),
        preferred_element_type=jnp.float32)
    sq = norms_ref[0:1, :] + d
    bits = jax.lax.bitcast_convert_type(sq, jnp.int32)
    col = jax.lax.broadcasted_iota(jnp.int32, (BQ, BK), 1)
    cand_ref[...] = jax.lax.bitcast_convert_type((bits & MASKHI) | col,
                                                 jnp.float32)

    def merge_chunk(c, carry):
        rows = pl.ds(c * RC, RC)
        g = [cand_ref[rows, i * 128:(i + 1) * 128] for i in range(NG)]
        for i, j in _NET8:
            lo = jnp.minimum(g[i], g[j])
            hi = jnp.maximum(g[i], g[j])
            g[i], g[j] = lo, hi
        r = [run_ref[qi, rows, i * 128:(i + 1) * 128] for i in range(NN)]
        cc = [r[0]] + [jnp.minimum(r[i], g[NN - 1 - i]) for i in range(1, NN)]
        for i, j in _NET13:
            lo = jnp.minimum(cc[i], cc[j])
            hi = jnp.maximum(cc[i], cc[j])
            cc[i], cc[j] = lo, hi

        @pl.when(k < NKB - 1)
        def _store():
            for i in range(NN):
                run_ref[qi, rows, i * 128:(i + 1) * 128] = cc[i]

        @pl.when(k == NKB - 1)
        def _extract():
            lvl = list(cc)
            outs = []
            for _ in range(NN):
                mn = jnp.min(lvl[0], axis=1, keepdims=True)
                won = lvl[0] == mn
                for i in range(NN - 1):
                    lvl[i] = jnp.where(won, lvl[i + 1], lvl[i])
                lvl[NN - 1] = jnp.where(won, BIGF, lvl[NN - 1])
                outs.append(mn)
            keys = jnp.concatenate(outs, axis=1)
            vb = jax.lax.bitcast_convert_type(keys, jnp.int32) & MASKHI
            vals = jax.lax.bitcast_convert_type(vb, jnp.float32)
            emb = emb_ref[rows, :]
            q_sq = jnp.sum(emb * emb, axis=1, keepdims=True)
            out_ref[rows, :] = jnp.sqrt(jnp.maximum(vals + q_sq, 1e-12))

        return carry

    jax.lax.fori_loop(0, BQ // RC, merge_chunk, 0, unroll=True)


@jax.jit
def kernel(embedding, memory_bank):
    return pl.pallas_call(
        _knn_body,
        grid=(NKB, NQB),
        in_specs=[
            pl.BlockSpec((BQ, D), lambda k, q: (q, 0)),
            pl.BlockSpec((BK, D), lambda k, q: (k, 0)),
        ],
        out_specs=pl.BlockSpec((BQ, NN), lambda k, q: (q, 0)),
        out_shape=jax.ShapeDtypeStruct((Q, NN), jnp.float32),
        scratch_shapes=[
            pltpu.VMEM((BQ, BK), jnp.float32),
            pltpu.VMEM((NQB, BQ, NN * 128), jnp.float32),
            pltpu.VMEM((8, BK), jnp.float32),
            pltpu.VMEM((BK, D), jnp.bfloat16),
            pltpu.VMEM((NQB, BQ, D), jnp.bfloat16),
        ],
        compiler_params=pltpu.CompilerParams(
            dimension_semantics=("arbitrary", "arbitrary")),
    )(embedding, memory_bank)
